# baseline (speedup 1.0000x reference)
import jax
import jax.numpy as jnp
from jax.experimental import pallas as pl

N = 100000
D = 512
G = 4096
OUT = 128
CW = 16
MB = 512


def _mlp_body(sums_ref, cnt_ref, w1, b1, w2, b2, w3, b3, out_ref):
    cnt = cnt_ref[:, :1]
    pooled = jnp.where(cnt > 0.5, sums_ref[...] / jnp.maximum(cnt, 1.0), 0.0)
    h = jnp.dot(pooled, w1[...], preferred_element_type=jnp.float32) + b1[...]
    h = jnp.maximum(h, 0.0)
    h = jnp.dot(h, w2[...], preferred_element_type=jnp.float32) + b2[...]
    h = jnp.maximum(h, 0.0)
    out_ref[...] = (jnp.dot(h, w3[...], preferred_element_type=jnp.float32)
                    + b3[...])


def _mlp(sums, cnts, w1t, b1, w2t, b2, w3t, b3):
    full = lambda shape: pl.BlockSpec(shape, lambda i: (0, 0))
    return pl.pallas_call(
        _mlp_body,
        grid=(G // MB,),
        in_specs=[
            pl.BlockSpec((MB, D), lambda i: (i, 0)),
            pl.BlockSpec((MB, CW), lambda i: (i, 0)),
            full((D, D // 2)),
            full((1, D // 2)),
            full((D // 2, D // 4)),
            full((1, D // 4)),
            full((D // 4, OUT)),
            full((1, OUT)),
        ],
        out_specs=pl.BlockSpec((MB, OUT), lambda i: (i, 0)),
        out_shape=jax.ShapeDtypeStruct((G, OUT), jnp.float32),
    )(sums, cnts, w1t, b1, w2t, b2, w3t, b3)


def kernel(x, batch, W1, b1, W2, b2, W3, b3):
    ids = batch.astype(jnp.int32)
    sums = jax.ops.segment_sum(x, ids, num_segments=G)
    counts = jax.ops.segment_sum(jnp.ones((N,), jnp.float32), ids,
                                 num_segments=G)
    cnts = jnp.broadcast_to(counts[:, None], (G, CW))
    return _mlp(sums, cnts, W1.T, b1[None, :], W2.T, b2[None, :],
                W3.T, b3[None, :])
